# baseline (speedup 1.0000x reference)
import functools

import jax
from jax import lax
from jax.experimental import pallas as pl
from jax.experimental.pallas import tpu as pltpu
from jax.experimental.pallas import tpu_sc as plsc

_CHUNK_ROWS = 16
_NCHUNK_PATH = 4


def kernel(x, emb_weight):
    seq_len = x.shape[1]
    dim = emb_weight.shape[1]
    info = plsc.get_sparse_core_info()
    num_cores = info.num_cores
    ns = info.num_subcores
    nw = num_cores * ns
    rows_per_w = seq_len // nw
    path_rows = _NCHUNK_PATH * _CHUNK_ROWS
    mesh = plsc.VectorSubcoreMesh(
        core_axis_name="c", subcore_axis_name="s", num_cores=num_cores
    )

    @functools.partial(
        pl.kernel,
        mesh=mesh,
        out_type=jax.ShapeDtypeStruct((seq_len, dim), emb_weight.dtype),
        scratch_types=[
            pltpu.VMEM((_NCHUNK_PATH, _CHUNK_ROWS, dim), emb_weight.dtype),
            pltpu.VMEM_SHARED(
                (ns, _NCHUNK_PATH, _CHUNK_ROWS, dim), emb_weight.dtype
            ),
        ]
        + [pltpu.SemaphoreType.DMA] * (4 * _NCHUNK_PATH),
    )
    def copy_k(table_hbm, out_hbm, buf_a, shared, *sems):
        sid = lax.axis_index("s")
        wid = sid * num_cores + lax.axis_index("c")
        base = wid * rows_per_w
        buf_b = shared.at[sid]
        in_a = sems[:_NCHUNK_PATH]
        in_b = sems[_NCHUNK_PATH : 2 * _NCHUNK_PATH]
        out_a = sems[2 * _NCHUNK_PATH : 3 * _NCHUNK_PATH]
        out_b = sems[3 * _NCHUNK_PATH :]

        def rows(path, i):
            return pl.ds(base + path * path_rows + i * _CHUNK_ROWS, _CHUNK_ROWS)

        gathers = []
        for i in range(_NCHUNK_PATH):
            gathers.append(
                pltpu.async_copy(table_hbm.at[rows(0, i)], buf_a.at[i], in_a[i])
            )
            gathers.append(
                pltpu.async_copy(table_hbm.at[rows(1, i)], buf_b.at[i], in_b[i])
            )
        scatters = []
        for i in range(_NCHUNK_PATH):
            gathers[2 * i].wait()
            scatters.append(
                pltpu.async_copy(buf_a.at[i], out_hbm.at[rows(0, i)], out_a[i])
            )
            gathers[2 * i + 1].wait()
            scatters.append(
                pltpu.async_copy(buf_b.at[i], out_hbm.at[rows(1, i)], out_b[i])
            )
        for s in scatters:
            s.wait()

    return copy_k(emb_weight)[None, :, :]
